# Initial kernel scaffold; baseline (speedup 1.0000x reference)
import jax
import jax.numpy as jnp
from jax.experimental import pallas as pl


def kernel(x, edge_index, W1, b1, W2, b2, Wlin, blin):
    raise NotImplementedError("write your pallas kernel here")


# speedup vs baseline: 8.9762x; 8.9762x over previous
import dataclasses
import functools

import jax
import jax.numpy as jnp
from jax import lax
from jax.experimental import pallas as pl
from jax.experimental.pallas import tpu as pltpu
from jax.experimental.pallas import tpu_sc as plsc

NS = 16
CH = 80
BM = 1024


def _mesh():
    return plsc.VectorSubcoreMesh(core_axis_name="c", subcore_axis_name="s")


@functools.lru_cache(maxsize=None)
def _deg_call(N: int, E: int):
    e_per_tile = E // (2 * NS)

    @functools.partial(
        pl.kernel,
        out_type=jax.ShapeDtypeStruct((2 * NS, N), jnp.float32),
        mesh=_mesh(),
        scratch_types=[
            pltpu.VMEM((CH,), jnp.int32),
            pltpu.VMEM((N,), jnp.float32),
        ],
        compiler_params=dataclasses.replace(pltpu.CompilerParams(),
                                            needs_layout_passes=False),
    )
    def deg_kernel(dst_hbm, out_hbm, idx_d, hist_v):
        cid = lax.axis_index("c")
        sid = lax.axis_index("s")
        wid = cid * NS + sid

        @pl.loop(0, N, step=16)
        def _(i):
            hist_v[pl.ds(i, 16)] = jnp.zeros((16,), jnp.float32)

        base = wid * e_per_tile
        ones_reg = jnp.full((16,), 1.0, jnp.float32)

        @pl.loop(0, e_per_tile, step=CH)
        def _(k):
            pltpu.sync_copy(dst_hbm.at[pl.ds(base + k, CH)], idx_d)
            for j in range(CH // 16):
                iv = idx_d[pl.ds(j * 16, 16)]
                plsc.addupdate_scatter(hist_v, [iv], ones_reg)

        pltpu.sync_copy(hist_v, out_hbm.at[wid])

    return deg_kernel


@functools.lru_cache(maxsize=None)
def _agg_call(N: int, E: int, H: int):
    e_per_tile = E // NS
    rpt = N // NS

    @functools.partial(
        pl.kernel,
        out_type=(jax.ShapeDtypeStruct((N, H), jnp.float32),
                  jax.ShapeDtypeStruct((N, H), jnp.float32)),
        mesh=_mesh(),
        scratch_types=[
            pltpu.VMEM((CH,), jnp.int32),
            pltpu.VMEM((CH,), jnp.int32),
            pltpu.VMEM((CH, H), jnp.float32),
            pltpu.VMEM_SHARED((N, H), jnp.float32),
            pltpu.SemaphoreType.DMA,
        ],
    )
    def agg_kernel(ylo_hbm, yhi_hbm, src_hbm, dst_hbm, z_hbm,
                   outlo_hbm, outhi_hbm, idx_s, idx_d, rows, acc, sem):
        cid = lax.axis_index("c")
        sid = lax.axis_index("s")
        pltpu.sync_copy(z_hbm, acc.at[pl.ds(sid * rpt, rpt)])
        plsc.subcore_barrier()
        tbase = sid * e_per_tile

        def process(y_hbm, out_hbm):
            @pl.loop(0, e_per_tile, step=CH)
            def _(k):
                b = tbase + k
                pltpu.sync_copy(src_hbm.at[pl.ds(b, CH)], idx_s)
                pltpu.sync_copy(dst_hbm.at[pl.ds(b, CH)], idx_d)
                pltpu.async_copy(y_hbm.at[idx_s], rows, sem).wait()
                pltpu.sync_copy(rows, acc.at[idx_d], add=True)

            plsc.subcore_barrier()
            pltpu.sync_copy(acc.at[pl.ds(sid * rpt, rpt)],
                            out_hbm.at[pl.ds(sid * rpt, rpt)])

        @pl.when(cid == 0)
        def _():
            process(ylo_hbm, outlo_hbm)

        @pl.when(cid == 1)
        def _():
            process(yhi_hbm, outhi_hbm)

    return agg_kernel


def _mm1_body(x_ref, w_ref, o_ref):
    o_ref[...] = jnp.dot(x_ref[...], w_ref[...],
                         preferred_element_type=jnp.float32)


@functools.lru_cache(maxsize=None)
def _mm1(N, K, C):
    return pl.pallas_call(
        _mm1_body,
        grid=(N // BM,),
        in_specs=[pl.BlockSpec((BM, K), lambda i: (i, 0)),
                  pl.BlockSpec((K, C), lambda i: (0, 0))],
        out_specs=pl.BlockSpec((BM, C), lambda i: (i, 0)),
        out_shape=jax.ShapeDtypeStruct((N, C), jnp.float32),
    )


def _e1_body(dg_ref, xw_ref, dis_ref, ylo_ref, yhi_ref):
    ones32 = jnp.ones((dg_ref.shape[0], 1), jnp.float32)
    cnt = lax.dot_general(dg_ref[...], ones32, (((0,), (0,)), ((), ())),
                          preferred_element_type=jnp.float32)
    dis = lax.rsqrt(1.0 + cnt)
    dis_ref[...] = dis
    yw = dis * xw_ref[...]
    h = yw.shape[1] // 2
    ylo_ref[...] = yw[:, :h]
    yhi_ref[...] = yw[:, h:]


@functools.lru_cache(maxsize=None)
def _e1(N, C):
    H = C // 2
    return pl.pallas_call(
        _e1_body,
        grid=(N // BM,),
        in_specs=[pl.BlockSpec((2 * NS, BM), lambda i: (0, i)),
                  pl.BlockSpec((BM, C), lambda i: (i, 0))],
        out_specs=(pl.BlockSpec((BM, 1), lambda i: (i, 0)),
                   pl.BlockSpec((BM, H), lambda i: (i, 0)),
                   pl.BlockSpec((BM, H), lambda i: (i, 0))),
        out_shape=(jax.ShapeDtypeStruct((N, 1), jnp.float32),
                   jax.ShapeDtypeStruct((N, H), jnp.float32),
                   jax.ShapeDtypeStruct((N, H), jnp.float32)),
    )


def _k2_body(dis_ref, alo_ref, ahi_ref, xw_ref, b_ref, w_ref,
             xw2_ref, ylo_ref, yhi_ref):
    dis = dis_ref[...]
    agg = jnp.concatenate([alo_ref[...], ahi_ref[...]], axis=1)
    h = jnp.maximum(dis * agg + (dis * dis) * xw_ref[...] + b_ref[...], 0.0)
    xw2 = jnp.dot(h, w_ref[...], preferred_element_type=jnp.float32)
    xw2_ref[...] = xw2
    yw = dis * xw2
    hh = yw.shape[1] // 2
    ylo_ref[...] = yw[:, :hh]
    yhi_ref[...] = yw[:, hh:]


@functools.lru_cache(maxsize=None)
def _k2(N, C, C2):
    H = C // 2
    H2 = C2 // 2
    return pl.pallas_call(
        _k2_body,
        grid=(N // BM,),
        in_specs=[pl.BlockSpec((BM, 1), lambda i: (i, 0)),
                  pl.BlockSpec((BM, H), lambda i: (i, 0)),
                  pl.BlockSpec((BM, H), lambda i: (i, 0)),
                  pl.BlockSpec((BM, C), lambda i: (i, 0)),
                  pl.BlockSpec((1, C), lambda i: (0, 0)),
                  pl.BlockSpec((C, C2), lambda i: (0, 0))],
        out_specs=(pl.BlockSpec((BM, C2), lambda i: (i, 0)),
                   pl.BlockSpec((BM, H2), lambda i: (i, 0)),
                   pl.BlockSpec((BM, H2), lambda i: (i, 0))),
        out_shape=(jax.ShapeDtypeStruct((N, C2), jnp.float32),
                   jax.ShapeDtypeStruct((N, H2), jnp.float32),
                   jax.ShapeDtypeStruct((N, H2), jnp.float32)),
    )


def _k3_body(dis_ref, alo_ref, ahi_ref, xw_ref, b_ref, w_ref, blin_ref,
             o_ref):
    dis = dis_ref[...]
    agg = jnp.concatenate([alo_ref[...], ahi_ref[...]], axis=1)
    h = jnp.maximum(dis * agg + (dis * dis) * xw_ref[...] + b_ref[...], 0.0)
    o_ref[...] = jnp.dot(h, w_ref[...],
                         preferred_element_type=jnp.float32) + blin_ref[...]


@functools.lru_cache(maxsize=None)
def _k3(N, C, O):
    H = C // 2
    return pl.pallas_call(
        _k3_body,
        grid=(N // BM,),
        in_specs=[pl.BlockSpec((BM, 1), lambda i: (i, 0)),
                  pl.BlockSpec((BM, H), lambda i: (i, 0)),
                  pl.BlockSpec((BM, H), lambda i: (i, 0)),
                  pl.BlockSpec((BM, C), lambda i: (i, 0)),
                  pl.BlockSpec((1, C), lambda i: (0, 0)),
                  pl.BlockSpec((C, O), lambda i: (0, 0)),
                  pl.BlockSpec((1, O), lambda i: (0, 0))],
        out_specs=pl.BlockSpec((BM, O), lambda i: (i, 0)),
        out_shape=jax.ShapeDtypeStruct((N, O), jnp.float32),
    )


def kernel(x, edge_index, W1, b1, W2, b2, Wlin, blin):
    N, Cin = x.shape
    E = edge_index.shape[1]
    C = W1.shape[1]
    C2 = W2.shape[1]
    O = Wlin.shape[1]
    H = C // 2

    NP = ((N + BM - 1) // BM) * BM

    xp = jnp.pad(x, ((0, NP - N), (0, 0)))
    src = edge_index[0].astype(jnp.int32)
    dst = edge_index[1].astype(jnp.int32)
    z128 = jnp.zeros((NP // NS, H), jnp.float32)

    degp = _deg_call(NP, E)(dst)
    xw1 = _mm1(NP, Cin, C)(xp, W1)
    dis, y1lo, y1hi = _e1(NP, C)(degp, xw1)
    a1lo, a1hi = _agg_call(NP, E, H)(y1lo, y1hi, src, dst, z128)
    xw2, y2lo, y2hi = _k2(NP, C, C2)(dis, a1lo, a1hi, xw1,
                                     b1.reshape(1, -1), W2)
    a2lo, a2hi = _agg_call(NP, E, C2 // 2)(y2lo, y2hi, src, dst, z128)
    out = _k3(NP, C2, O)(dis, a2lo, a2hi, xw2,
                         b2.reshape(1, -1), Wlin, blin.reshape(1, -1))
    return out[:N]
